# baseline (speedup 1.0000x reference)
import jax
import jax.numpy as jnp
from jax import lax
from jax.experimental import pallas as pl
from jax.experimental.pallas import tpu as pltpu

_NUM_PATCHES = 196
_EMBED_DIM = 768
_HIDDEN_DIM = 512
_PATCH_SIZE = 16
_B = 16
_LEN_KEEP = 49

_K_TOTAL = _NUM_PATCHES * _EMBED_DIM
_K_CHUNK = 18816
_K_BLK = 1568
_STEPS_PER_CHUNK = _K_CHUNK // _K_BLK
_GRID = _K_TOTAL // _K_BLK


def _fused_body(unet_ref, a_ref, w1_ref, b1_ref, w2_ref, b2_ref, patches_ref,
                xm_ref, mask_ref, idr_ref, mp_ref, wbf_ref, acc_ref):
    k = pl.program_id(0)

    @pl.when(k == 0)
    def _init():
        acc_ref[...] = jnp.zeros_like(acc_ref)

    j = lax.rem(k, _STEPS_PER_CHUNK)
    wbf_ref[pl.ds(j * _K_BLK, _K_BLK), :] = w1_ref[...].astype(jnp.bfloat16)

    @pl.when(j == _STEPS_PER_CHUNK - 1)
    def _accumulate():
        acc_ref[...] += jnp.dot(a_ref[...], wbf_ref[...],
                                preferred_element_type=jnp.float32)

    @pl.when(k == pl.num_programs(0) - 1)
    def _epilogue():
        h = jnp.maximum(acc_ref[...] + b1_ref[...], 0.0).astype(jnp.bfloat16)
        logits = jnp.dot(h, w2_ref[...],
                         preferred_element_type=jnp.float32) + b2_ref[...]
        learned = jax.nn.sigmoid(logits)
        mp = 0.7 * unet_ref[...] + 0.3 * learned
        mp_ref[...] = mp

        vi = mp[:, :, None]
        vj = mp[:, None, :]
        row_id = lax.broadcasted_iota(jnp.int32, (_B, _NUM_PATCHES, _NUM_PATCHES), 1)
        col_id = lax.broadcasted_iota(jnp.int32, (_B, _NUM_PATCHES, _NUM_PATCHES), 2)
        cnt = (vj < vi) | ((vj == vi) & (col_id < row_id))
        rank = jnp.sum(cnt.astype(jnp.int32), axis=2)
        idr_ref[...] = rank
        mask_ref[...] = (rank >= _LEN_KEEP).astype(jnp.float32)

        keep_slot = lax.broadcasted_iota(jnp.int32, (_B, _LEN_KEEP, _NUM_PATCHES), 1)
        onehot = (rank[:, None, :] == keep_slot).astype(jnp.float32)
        xm_ref[...] = lax.dot_general(
            onehot, patches_ref[...],
            dimension_numbers=(((2,), (1,)), ((0,), (0,))),
            preferred_element_type=jnp.float32,
            precision=lax.Precision.HIGHEST)


def kernel(images, patches, W1, b1, W2, b2):
    Bq, N, D = patches.shape
    seg_masks = jax.random.uniform(
        jax.random.key(42), (Bq, 1, images.shape[2], images.shape[3]),
        dtype=jnp.float32)
    nph = images.shape[2] // _PATCH_SIZE
    npw = images.shape[3] // _PATCH_SIZE
    p = seg_masks.reshape(Bq, 1, nph, _PATCH_SIZE, npw, _PATCH_SIZE)
    unet = 1.0 - p.mean(axis=(1, 3, 5)).reshape(Bq, -1)

    a_bf = patches.reshape(Bq, N * D).astype(jnp.bfloat16)

    out_shapes = (
        jax.ShapeDtypeStruct((Bq, _LEN_KEEP, D), jnp.float32),
        jax.ShapeDtypeStruct((Bq, N), jnp.float32),
        jax.ShapeDtypeStruct((Bq, N), jnp.int32),
        jax.ShapeDtypeStruct((Bq, N), jnp.float32),
    )
    x_masked, mask, ids_restore, mask_prob = pl.pallas_call(
        _fused_body,
        grid=(_GRID,),
        in_specs=[
            pl.BlockSpec((Bq, N), lambda k: (0, 0)),
            pl.BlockSpec((Bq, _K_CHUNK), lambda k: (0, k // _STEPS_PER_CHUNK)),
            pl.BlockSpec((_K_BLK, _HIDDEN_DIM), lambda k: (k, 0)),
            pl.BlockSpec((1, _HIDDEN_DIM), lambda k: (0, 0)),
            pl.BlockSpec((_HIDDEN_DIM, N), lambda k: (0, 0)),
            pl.BlockSpec((1, N), lambda k: (0, 0)),
            pl.BlockSpec((Bq, N, D), lambda k: (0, 0, 0)),
        ],
        out_specs=(
            pl.BlockSpec((Bq, _LEN_KEEP, D), lambda k: (0, 0, 0)),
            pl.BlockSpec((Bq, N), lambda k: (0, 0)),
            pl.BlockSpec((Bq, N), lambda k: (0, 0)),
            pl.BlockSpec((Bq, N), lambda k: (0, 0)),
        ),
        out_shape=out_shapes,
        scratch_shapes=[
            pltpu.VMEM((_K_CHUNK, _HIDDEN_DIM), jnp.bfloat16),
            pltpu.VMEM((Bq, _HIDDEN_DIM), jnp.float32),
        ],
        compiler_params=pltpu.CompilerParams(
            dimension_semantics=("arbitrary",),
        ),
    )(unet, a_bf, W1, b1.reshape(1, -1), W2, b2.reshape(1, -1), patches)
    return (x_masked, mask, ids_restore, mask_prob)


# speedup vs baseline: 1.0750x; 1.0750x over previous
import jax
import jax.numpy as jnp
from jax import lax
from jax.experimental import pallas as pl
from jax.experimental.pallas import tpu as pltpu

_NUM_PATCHES = 196
_EMBED_DIM = 768
_HIDDEN_DIM = 512
_PATCH_SIZE = 16
_B = 16
_LEN_KEEP = 49

_K_TOTAL = _NUM_PATCHES * _EMBED_DIM
_K_CHUNK = 18816
_K_BLK = 1568
_BLKS_PER_STEP = 2
_STEPS_PER_CHUNK = _K_CHUNK // (_K_BLK * _BLKS_PER_STEP)
_GRID = _K_TOTAL // (_K_BLK * _BLKS_PER_STEP)


def _fused_body(unet_ref, a_ref, w1a_ref, w1b_ref, b1_ref, w2_ref, b2_ref,
                patches_ref, xm_ref, mask_ref, idr_ref, mp_ref,
                wbf_ref, acc_ref):
    k = pl.program_id(0)

    @pl.when(k == 0)
    def _init():
        acc_ref[...] = jnp.zeros_like(acc_ref)

    j = lax.rem(k, _STEPS_PER_CHUNK)
    base = j * (_K_BLK * _BLKS_PER_STEP)
    wbf_ref[pl.ds(base, _K_BLK), :] = w1a_ref[...].astype(jnp.bfloat16)
    wbf_ref[pl.ds(base + _K_BLK, _K_BLK), :] = w1b_ref[...].astype(jnp.bfloat16)

    @pl.when(j == _STEPS_PER_CHUNK - 1)
    def _accumulate():
        acc_ref[...] += jnp.dot(a_ref[...], wbf_ref[...],
                                preferred_element_type=jnp.float32)

    @pl.when(k == pl.num_programs(0) - 1)
    def _epilogue():
        h = jnp.maximum(acc_ref[...] + b1_ref[...], 0.0).astype(jnp.bfloat16)
        logits = jnp.dot(h, w2_ref[...],
                         preferred_element_type=jnp.float32) + b2_ref[...]
        learned = jax.nn.sigmoid(logits)
        mp = 0.7 * unet_ref[...] + 0.3 * learned
        mp_ref[...] = mp

        vi = mp[:, :, None]
        vj = mp[:, None, :]
        row_id = lax.broadcasted_iota(jnp.int32, (_B, _NUM_PATCHES, _NUM_PATCHES), 1)
        col_id = lax.broadcasted_iota(jnp.int32, (_B, _NUM_PATCHES, _NUM_PATCHES), 2)
        cnt = (vj < vi) | ((vj == vi) & (col_id < row_id))
        rank = jnp.sum(cnt.astype(jnp.int32), axis=2)
        idr_ref[...] = rank
        mask_ref[...] = (rank >= _LEN_KEEP).astype(jnp.float32)

        keep_slot = lax.broadcasted_iota(jnp.int32, (_LEN_KEEP, _NUM_PATCHES), 0)
        for b in range(_B):
            onehot = (rank[b:b + 1, :] == keep_slot).astype(jnp.float32)
            xm_ref[b] = jnp.dot(onehot, patches_ref[b],
                                preferred_element_type=jnp.float32,
                                precision=lax.Precision.HIGHEST)


def kernel(images, patches, W1, b1, W2, b2):
    Bq, N, D = patches.shape
    seg_masks = jax.random.uniform(
        jax.random.key(42), (Bq, 1, images.shape[2], images.shape[3]),
        dtype=jnp.float32)
    nph = images.shape[2] // _PATCH_SIZE
    npw = images.shape[3] // _PATCH_SIZE
    p = seg_masks.reshape(Bq, 1, nph, _PATCH_SIZE, npw, _PATCH_SIZE)
    unet = 1.0 - p.mean(axis=(1, 3, 5)).reshape(Bq, -1)

    a_bf = patches.reshape(Bq, N * D).astype(jnp.bfloat16)

    out_shapes = (
        jax.ShapeDtypeStruct((Bq, _LEN_KEEP, D), jnp.float32),
        jax.ShapeDtypeStruct((Bq, N), jnp.float32),
        jax.ShapeDtypeStruct((Bq, N), jnp.int32),
        jax.ShapeDtypeStruct((Bq, N), jnp.float32),
    )
    x_masked, mask, ids_restore, mask_prob = pl.pallas_call(
        _fused_body,
        grid=(_GRID,),
        in_specs=[
            pl.BlockSpec((Bq, N), lambda k: (0, 0)),
            pl.BlockSpec((Bq, _K_CHUNK), lambda k: (0, k // _STEPS_PER_CHUNK)),
            pl.BlockSpec((_K_BLK, _HIDDEN_DIM), lambda k: (2 * k, 0)),
            pl.BlockSpec((_K_BLK, _HIDDEN_DIM), lambda k: (2 * k + 1, 0)),
            pl.BlockSpec((1, _HIDDEN_DIM), lambda k: (0, 0)),
            pl.BlockSpec((_HIDDEN_DIM, N), lambda k: (0, 0)),
            pl.BlockSpec((1, N), lambda k: (0, 0)),
            pl.BlockSpec((Bq, N, D), lambda k: (0, 0, 0)),
        ],
        out_specs=(
            pl.BlockSpec((Bq, _LEN_KEEP, D), lambda k: (0, 0, 0)),
            pl.BlockSpec((Bq, N), lambda k: (0, 0)),
            pl.BlockSpec((Bq, N), lambda k: (0, 0)),
            pl.BlockSpec((Bq, N), lambda k: (0, 0)),
        ),
        out_shape=out_shapes,
        scratch_shapes=[
            pltpu.VMEM((_K_CHUNK, _HIDDEN_DIM), jnp.bfloat16),
            pltpu.VMEM((Bq, _HIDDEN_DIM), jnp.float32),
        ],
        compiler_params=pltpu.CompilerParams(
            dimension_semantics=("arbitrary",),
        ),
    )(unet, a_bf, W1, W1, b1.reshape(1, -1), W2, b2.reshape(1, -1), patches)
    return (x_masked, mask, ids_restore, mask_prob)


# speedup vs baseline: 1.0759x; 1.0008x over previous
import jax
import jax.numpy as jnp
from jax import lax
from jax.experimental import pallas as pl
from jax.experimental.pallas import tpu as pltpu

_NUM_PATCHES = 196
_EMBED_DIM = 768
_HIDDEN_DIM = 512
_PATCH_SIZE = 16
_B = 16
_LEN_KEEP = 49

_K_TOTAL = _NUM_PATCHES * _EMBED_DIM
_K_CHUNK = 18816
_K_BLK = 1568
_BLKS_PER_STEP = 2
_STEPS_PER_CHUNK = _K_CHUNK // (_K_BLK * _BLKS_PER_STEP)
_GRID = _K_TOTAL // (_K_BLK * _BLKS_PER_STEP)


def _fused_body(unet_ref, a_ref, w1a_ref, w1b_ref, b1_ref, w2_ref, b2_ref,
                patches_ref, xm_ref, mask_ref, idr_ref, mp_ref,
                wbf_ref, acc_ref):
    k = pl.program_id(0)

    @pl.when(k == 0)
    def _init():
        acc_ref[...] = jnp.zeros_like(acc_ref)

    j = lax.rem(k, _STEPS_PER_CHUNK)
    base = j * (_K_BLK * _BLKS_PER_STEP)
    wbf_ref[pl.ds(base, _K_BLK), :] = w1a_ref[...].astype(jnp.bfloat16)
    wbf_ref[pl.ds(base + _K_BLK, _K_BLK), :] = w1b_ref[...].astype(jnp.bfloat16)

    @pl.when(j == _STEPS_PER_CHUNK - 1)
    def _accumulate():
        acc_ref[...] += jnp.dot(a_ref[...], wbf_ref[...],
                                preferred_element_type=jnp.float32)

    @pl.when(k == pl.num_programs(0) - 1)
    def _epilogue():
        h = jnp.maximum(acc_ref[...] + b1_ref[...], 0.0).astype(jnp.bfloat16)
        w2 = w2_ref[...]
        logits = (jnp.dot(h[:, :256], w2[:256], preferred_element_type=jnp.float32)
                  + jnp.dot(h[:, 256:], w2[256:], preferred_element_type=jnp.float32)
                  ) + b2_ref[...]
        learned = jax.nn.sigmoid(logits)
        mp = 0.7 * unet_ref[...] + 0.3 * learned
        mp_ref[...] = mp

        vi = mp[:, :, None]
        vj = mp[:, None, :]
        row_id = lax.broadcasted_iota(jnp.int32, (_B, _NUM_PATCHES, _NUM_PATCHES), 1)
        col_id = lax.broadcasted_iota(jnp.int32, (_B, _NUM_PATCHES, _NUM_PATCHES), 2)
        cnt = (vj < vi) | ((vj == vi) & (col_id < row_id))
        rank = jnp.sum(cnt.astype(jnp.int32), axis=2)
        idr_ref[...] = rank
        mask_ref[...] = (rank >= _LEN_KEEP).astype(jnp.float32)

        keep_slot = lax.broadcasted_iota(jnp.int32, (_LEN_KEEP, _NUM_PATCHES), 0)
        for b in range(_B):
            onehot = (rank[b:b + 1, :] == keep_slot).astype(jnp.float32)
            xm_ref[b] = jnp.dot(onehot, patches_ref[b],
                                preferred_element_type=jnp.float32,
                                precision=lax.Precision.HIGHEST)


def kernel(images, patches, W1, b1, W2, b2):
    Bq, N, D = patches.shape
    seg_masks = jax.random.uniform(
        jax.random.key(42), (Bq, 1, images.shape[2], images.shape[3]),
        dtype=jnp.float32)
    nph = images.shape[2] // _PATCH_SIZE
    npw = images.shape[3] // _PATCH_SIZE
    p = seg_masks.reshape(Bq, 1, nph, _PATCH_SIZE, npw, _PATCH_SIZE)
    unet = 1.0 - p.mean(axis=(1, 3, 5)).reshape(Bq, -1)

    a_bf = patches.reshape(Bq, N * D).astype(jnp.bfloat16)

    out_shapes = (
        jax.ShapeDtypeStruct((Bq, _LEN_KEEP, D), jnp.float32),
        jax.ShapeDtypeStruct((Bq, N), jnp.float32),
        jax.ShapeDtypeStruct((Bq, N), jnp.int32),
        jax.ShapeDtypeStruct((Bq, N), jnp.float32),
    )
    x_masked, mask, ids_restore, mask_prob = pl.pallas_call(
        _fused_body,
        grid=(_GRID,),
        in_specs=[
            pl.BlockSpec((Bq, N), lambda k: (0, 0)),
            pl.BlockSpec((Bq, _K_CHUNK), lambda k: (0, k // _STEPS_PER_CHUNK)),
            pl.BlockSpec((_K_BLK, _HIDDEN_DIM), lambda k: (2 * k, 0)),
            pl.BlockSpec((_K_BLK, _HIDDEN_DIM), lambda k: (2 * k + 1, 0)),
            pl.BlockSpec((1, _HIDDEN_DIM), lambda k: (0, 0)),
            pl.BlockSpec((_HIDDEN_DIM, N), lambda k: (0, 0)),
            pl.BlockSpec((1, N), lambda k: (0, 0)),
            pl.BlockSpec((Bq, N, D), lambda k: (0, 0, 0)),
        ],
        out_specs=(
            pl.BlockSpec((Bq, _LEN_KEEP, D), lambda k: (0, 0, 0)),
            pl.BlockSpec((Bq, N), lambda k: (0, 0)),
            pl.BlockSpec((Bq, N), lambda k: (0, 0)),
            pl.BlockSpec((Bq, N), lambda k: (0, 0)),
        ),
        out_shape=out_shapes,
        scratch_shapes=[
            pltpu.VMEM((_K_CHUNK, _HIDDEN_DIM), jnp.bfloat16),
            pltpu.VMEM((Bq, _HIDDEN_DIM), jnp.float32),
        ],
        compiler_params=pltpu.CompilerParams(
            dimension_semantics=("arbitrary",),
        ),
    )(unet, a_bf, W1, W1, b1.reshape(1, -1), W2, b2.reshape(1, -1), patches)
    return (x_masked, mask, ids_restore, mask_prob)
